# baseline (speedup 1.0000x reference)
import functools
import numpy as np
import jax
import jax.numpy as jnp
from jax import lax
from jax.experimental import pallas as pl
from jax.experimental.pallas import tpu as pltpu
from jax.experimental.pallas import tpu_sc as plsc

_Q = 0.7
_SEED = 42
_C = 128
_NC = 2
_NS = 16
_NW = _NC * _NS

_plan_cache = {}


def _threefry2x32(k0, k1, x0, x1):
    rot = (13, 15, 26, 6, 17, 29, 16, 24)
    ks = (np.uint32(k0), np.uint32(k1),
          np.uint32(k0) ^ np.uint32(k1) ^ np.uint32(0x1BD11BDA))
    x0 = (x0 + ks[0]).astype(np.uint32)
    x1 = (x1 + ks[1]).astype(np.uint32)
    for i in range(5):
        for r in rot[:4] if i % 2 == 0 else rot[4:]:
            x0 = (x0 + x1).astype(np.uint32)
            x1 = ((x1 << np.uint32(r)) | (x1 >> np.uint32(32 - r))).astype(np.uint32)
            x1 = x1 ^ x0
        x0 = (x0 + ks[(i + 1) % 3]).astype(np.uint32)
        x1 = (x1 + ks[(i + 2) % 3] + np.uint32(i + 1)).astype(np.uint32)
    return x0, x1


def _bernoulli_mask(seed, p, n):
    k0 = np.uint32(np.uint64(seed) >> np.uint64(32))
    k1 = np.uint32(np.uint64(seed) & np.uint64(0xFFFFFFFF))
    idx = np.arange(n, dtype=np.uint64)
    c1 = (idx >> np.uint64(32)).astype(np.uint32)
    c2 = (idx & np.uint64(0xFFFFFFFF)).astype(np.uint32)
    b1, b2 = _threefry2x32(k0, k1, c1, c2)
    bits = b1 ^ b2
    floats = ((bits >> np.uint32(9)) | np.uint32(0x3F800000)).view(np.float32)
    u = np.maximum(np.float32(0), floats - np.float32(1))
    return u < np.float32(p)


def _pack(idx):
    per = -(-len(idx) // _NW)
    per = -(-per // _C) * _C
    npad = per * _NW - len(idx)
    reps = -(-npad // len(idx))
    pad = np.tile(idx, reps)[:npad]
    return np.concatenate([idx, pad]).reshape(_NW, per // _C, _C)


def _mask_plan(n):
    if n not in _plan_cache:
        mask = _bernoulli_mask(_SEED, _Q, n)
        keep = np.nonzero(~mask)[0].astype(np.int32)
        zero = np.nonzero(mask)[0].astype(np.int32)
        _plan_cache[n] = (_pack(keep), _pack(zero))
    return _plan_cache[n]


def _sc_body(kc, zc, gr, x_hbm, kflat_hbm, kidx_hbm, zidx_hbm, zeros_hbm,
             out_hbm, kflat_v, kidx_v, zidx_v, rowbuf, zbuf, gsem, ssem, zsem):
    kc2 = kc // gr
    w = lax.axis_index("s") * _NC + lax.axis_index("c")
    pltpu.sync_copy(kflat_hbm.at[w], kflat_v)
    pltpu.sync_copy(kidx_hbm.at[w], kidx_v)
    pltpu.sync_copy(zidx_hbm.at[w], zidx_v)
    pltpu.sync_copy(zeros_hbm.at[w], zbuf)

    zd = [pltpu.async_copy(zbuf, out_hbm.at[zidx_v.at[j]], zsem)
          for j in range(zc)]

    def gather(t):
        return pltpu.async_copy(
            x_hbm.at[kflat_v.at[pl.ds(t * gr * _C, gr * _C)]],
            rowbuf.at[t % 2], gsem)

    def scatter(t, r):
        return pltpu.async_copy(
            rowbuf.at[t % 2, pl.ds(r * _C, _C)],
            out_hbm.at[kidx_v.at[t * gr + r]], ssem)

    gd = {0: gather(0)}
    if kc2 > 1:
        gd[1] = gather(1)
    sd = {}
    for t in range(kc2):
        gd[t].wait()
        sd[t] = [scatter(t, r) for r in range(gr)]
        if t + 2 < kc2:
            for s in sd[t]:
                s.wait()
            sd[t] = []
            gd[t + 2] = gather(t + 2)
    for t in range(kc2):
        for s in sd[t]:
            s.wait()
    for d in zd:
        d.wait()


def kernel(x):
    n, d = x.shape
    kidx, zidx = _mask_plan(n)
    kc, zc = kidx.shape[1], zidx.shape[1]
    gr = 2 if kc % 2 == 0 else 1
    zeros = jnp.zeros((_NW, _C, d), x.dtype)
    kflat = kidx.reshape(_NW, kc * _C)
    mesh = plsc.VectorSubcoreMesh(
        core_axis_name="c", subcore_axis_name="s",
        num_cores=_NC, num_subcores=_NS)
    body = functools.partial(_sc_body, kc, zc, gr)
    return pl.kernel(
        body,
        out_type=jax.ShapeDtypeStruct((n, d), x.dtype),
        mesh=mesh,
        scratch_types=[
            pltpu.VMEM((kc * _C,), jnp.int32),
            pltpu.VMEM((kc, _C), jnp.int32),
            pltpu.VMEM((zc, _C), jnp.int32),
            pltpu.VMEM((2, gr * _C, d), x.dtype),
            pltpu.VMEM((_C, d), x.dtype),
            pltpu.SemaphoreType.DMA,
            pltpu.SemaphoreType.DMA,
            pltpu.SemaphoreType.DMA,
        ],
    )(x, jnp.asarray(kflat), jnp.asarray(kidx), jnp.asarray(zidx), zeros)


# speedup vs baseline: 1.0156x; 1.0156x over previous
import functools
import numpy as np
import jax
import jax.numpy as jnp
from jax import lax
from jax.experimental import pallas as pl
from jax.experimental.pallas import tpu as pltpu
from jax.experimental.pallas import tpu_sc as plsc

_Q = 0.7
_SEED = 42
_C = 128
_NC = 2
_NS = 16
_NW = _NC * _NS

_plan_cache = {}


def _threefry2x32(k0, k1, x0, x1):
    rot = (13, 15, 26, 6, 17, 29, 16, 24)
    ks = (np.uint32(k0), np.uint32(k1),
          np.uint32(k0) ^ np.uint32(k1) ^ np.uint32(0x1BD11BDA))
    x0 = (x0 + ks[0]).astype(np.uint32)
    x1 = (x1 + ks[1]).astype(np.uint32)
    for i in range(5):
        for r in rot[:4] if i % 2 == 0 else rot[4:]:
            x0 = (x0 + x1).astype(np.uint32)
            x1 = ((x1 << np.uint32(r)) | (x1 >> np.uint32(32 - r))).astype(np.uint32)
            x1 = x1 ^ x0
        x0 = (x0 + ks[(i + 1) % 3]).astype(np.uint32)
        x1 = (x1 + ks[(i + 2) % 3] + np.uint32(i + 1)).astype(np.uint32)
    return x0, x1


def _bernoulli_mask(seed, p, n):
    k0 = np.uint32(np.uint64(seed) >> np.uint64(32))
    k1 = np.uint32(np.uint64(seed) & np.uint64(0xFFFFFFFF))
    idx = np.arange(n, dtype=np.uint64)
    c1 = (idx >> np.uint64(32)).astype(np.uint32)
    c2 = (idx & np.uint64(0xFFFFFFFF)).astype(np.uint32)
    b1, b2 = _threefry2x32(k0, k1, c1, c2)
    bits = b1 ^ b2
    floats = ((bits >> np.uint32(9)) | np.uint32(0x3F800000)).view(np.float32)
    u = np.maximum(np.float32(0), floats - np.float32(1))
    return u < np.float32(p)


def _pack(idx):
    per = -(-len(idx) // _NW)
    per = -(-per // _C) * _C
    npad = per * _NW - len(idx)
    reps = -(-npad // len(idx))
    pad = np.tile(idx, reps)[:npad]
    return np.concatenate([idx, pad]).reshape(_NW, per // _C, _C)


def _mask_plan(n):
    if n not in _plan_cache:
        mask = _bernoulli_mask(_SEED, _Q, n)
        keep = np.nonzero(~mask)[0].astype(np.int32)
        zero = np.nonzero(mask)[0].astype(np.int32)
        _plan_cache[n] = (_pack(keep), _pack(zero))
    return _plan_cache[n]


def _sc_body(kc, zc, x_hbm, kidx_hbm, zidx_hbm, zeros_hbm, out_hbm,
             kidx_v, zidx_v, rowbuf, zbuf, gsem, ssem, zsem):
    w = lax.axis_index("s") * _NC + lax.axis_index("c")
    pltpu.sync_copy(kidx_hbm.at[w], kidx_v)
    pltpu.sync_copy(zidx_hbm.at[w], zidx_v)
    pltpu.sync_copy(zeros_hbm.at[w], zbuf)

    def gather(j):
        return pltpu.async_copy(x_hbm.at[kidx_v.at[j]], rowbuf.at[j % 2], gsem)

    def scatter(j):
        return pltpu.async_copy(rowbuf.at[j % 2], out_hbm.at[kidx_v.at[j]], ssem)

    def zero(j):
        return pltpu.async_copy(zbuf, out_hbm.at[zidx_v.at[j]], zsem)

    zq = [zero(0), zero(1)]
    znext = 2
    gd = {0: gather(0)}
    sd = {}
    for j in range(kc):
        gd[j].wait()
        if j + 1 < kc:
            if j - 1 >= 0:
                sd[j - 1].wait()
            gd[j + 1] = gather(j + 1)
        sd[j] = scatter(j)
        take = -(-(zc - znext) // max(1, kc - 1 - j))
        for _ in range(take):
            if znext < zc:
                zq.append(zero(znext))
                znext += 1
    while znext < zc:
        zq.append(zero(znext))
        znext += 1
    for j in range(max(0, kc - 2), kc):
        sd[j].wait()
    for d in zq:
        d.wait()


def kernel(x):
    n, d = x.shape
    kidx, zidx = _mask_plan(n)
    kc, zc = kidx.shape[1], zidx.shape[1]
    zeros = jnp.zeros((_NW, _C, d), x.dtype)
    mesh = plsc.VectorSubcoreMesh(
        core_axis_name="c", subcore_axis_name="s",
        num_cores=_NC, num_subcores=_NS)
    body = functools.partial(_sc_body, kc, zc)
    return pl.kernel(
        body,
        out_type=jax.ShapeDtypeStruct((n, d), x.dtype),
        mesh=mesh,
        scratch_types=[
            pltpu.VMEM((kc, _C), jnp.int32),
            pltpu.VMEM((zc, _C), jnp.int32),
            pltpu.VMEM((2, _C, d), x.dtype),
            pltpu.VMEM((_C, d), x.dtype),
            pltpu.SemaphoreType.DMA,
            pltpu.SemaphoreType.DMA,
            pltpu.SemaphoreType.DMA,
        ],
    )(x, jnp.asarray(kidx), jnp.asarray(zidx), zeros)


# speedup vs baseline: 1.0879x; 1.0712x over previous
import functools
import numpy as np
import jax
import jax.numpy as jnp
from jax import lax
from jax.experimental import pallas as pl
from jax.experimental.pallas import tpu as pltpu
from jax.experimental.pallas import tpu_sc as plsc

_Q = 0.7
_SEED = 42
_C = 128
_NC = 2
_NS = 16
_NW = _NC * _NS

_plan_cache = {}


def _threefry2x32(k0, k1, x0, x1):
    rot = (13, 15, 26, 6, 17, 29, 16, 24)
    ks = (np.uint32(k0), np.uint32(k1),
          np.uint32(k0) ^ np.uint32(k1) ^ np.uint32(0x1BD11BDA))
    x0 = (x0 + ks[0]).astype(np.uint32)
    x1 = (x1 + ks[1]).astype(np.uint32)
    for i in range(5):
        for r in rot[:4] if i % 2 == 0 else rot[4:]:
            x0 = (x0 + x1).astype(np.uint32)
            x1 = ((x1 << np.uint32(r)) | (x1 >> np.uint32(32 - r))).astype(np.uint32)
            x1 = x1 ^ x0
        x0 = (x0 + ks[(i + 1) % 3]).astype(np.uint32)
        x1 = (x1 + ks[(i + 2) % 3] + np.uint32(i + 1)).astype(np.uint32)
    return x0, x1


def _bernoulli_mask(seed, p, n):
    k0 = np.uint32(np.uint64(seed) >> np.uint64(32))
    k1 = np.uint32(np.uint64(seed) & np.uint64(0xFFFFFFFF))
    idx = np.arange(n, dtype=np.uint64)
    c1 = (idx >> np.uint64(32)).astype(np.uint32)
    c2 = (idx & np.uint64(0xFFFFFFFF)).astype(np.uint32)
    b1, b2 = _threefry2x32(k0, k1, c1, c2)
    bits = b1 ^ b2
    floats = ((bits >> np.uint32(9)) | np.uint32(0x3F800000)).view(np.float32)
    u = np.maximum(np.float32(0), floats - np.float32(1))
    return u < np.float32(p)


def _pack(idx):
    per = -(-len(idx) // _NW)
    per = -(-per // _C) * _C
    npad = per * _NW - len(idx)
    reps = -(-npad // len(idx))
    pad = np.tile(idx, reps)[:npad]
    return np.concatenate([idx, pad]).reshape(_NW, per // _C, _C)


def _mask_plan(n):
    if n not in _plan_cache:
        mask = _bernoulli_mask(_SEED, _Q, n)
        keep = np.nonzero(~mask)[0].astype(np.int32)
        zero = np.nonzero(mask)[0].astype(np.int32)
        _plan_cache[n] = (_pack(keep), _pack(zero))
    return _plan_cache[n]


def _sc_body(kc, zc, x_hbm, kidx_hbm, zidx_hbm, zeros_hbm, out_hbm,
             kidx_v, zidx_v, rowbuf, zbuf, gsem, ssem, zsem):
    nb = rowbuf.shape[0]
    w = lax.axis_index("s") * _NC + lax.axis_index("c")
    pk = pltpu.async_copy(kidx_hbm.at[w], kidx_v, gsem)
    pz = pltpu.async_copy(zidx_hbm.at[w], zidx_v, ssem)
    pb = pltpu.async_copy(zeros_hbm.at[w], zbuf, zsem)

    def gather(j):
        return pltpu.async_copy(x_hbm.at[kidx_v.at[j]], rowbuf.at[j % nb], gsem)

    def scatter(j):
        return pltpu.async_copy(rowbuf.at[j % nb], out_hbm.at[kidx_v.at[j]], ssem)

    def zero(j):
        return pltpu.async_copy(zbuf, out_hbm.at[zidx_v.at[j]], zsem)

    pk.wait()
    gd = {j: gather(j) for j in range(min(nb, kc))}
    pz.wait()
    pb.wait()
    zq = [zero(0), zero(1)]
    znext = 2
    sd = {}
    for j in range(kc):
        gd[j].wait()
        sd[j] = scatter(j)
        if j >= 1 and j + nb - 1 < kc:
            sd[j - 1].wait()
            gd[j + nb - 1] = gather(j + nb - 1)
        take = -(-(zc - znext) // max(1, kc - 1 - j))
        for _ in range(take):
            if znext < zc:
                zq.append(zero(znext))
                znext += 1
    while znext < zc:
        zq.append(zero(znext))
        znext += 1
    for j in sorted(sd):
        if not (1 <= j + 1 and j + nb < kc):
            sd[j].wait()
    for d in zq:
        d.wait()


def kernel(x):
    n, d = x.shape
    kidx, zidx = _mask_plan(n)
    kc, zc = kidx.shape[1], zidx.shape[1]
    zeros = jnp.zeros((_NW, _C, d), x.dtype)
    mesh = plsc.VectorSubcoreMesh(
        core_axis_name="c", subcore_axis_name="s",
        num_cores=_NC, num_subcores=_NS)
    body = functools.partial(_sc_body, kc, zc)
    return pl.kernel(
        body,
        out_type=jax.ShapeDtypeStruct((n, d), x.dtype),
        mesh=mesh,
        scratch_types=[
            pltpu.VMEM((kc, _C), jnp.int32),
            pltpu.VMEM((zc, _C), jnp.int32),
            pltpu.VMEM((4, _C, d), x.dtype),
            pltpu.VMEM((_C, d), x.dtype),
            pltpu.SemaphoreType.DMA,
            pltpu.SemaphoreType.DMA,
            pltpu.SemaphoreType.DMA,
        ],
    )(x, jnp.asarray(kidx), jnp.asarray(zidx), zeros)


# speedup vs baseline: 1.1036x; 1.0144x over previous
import functools
import numpy as np
import jax
import jax.numpy as jnp
from jax import lax
from jax.experimental import pallas as pl
from jax.experimental.pallas import tpu as pltpu
from jax.experimental.pallas import tpu_sc as plsc

_Q = 0.7
_SEED = 42
_C = 128
_NC = 2
_NS = 16
_NW = _NC * _NS

_plan_cache = {}


def _threefry2x32(k0, k1, x0, x1):
    rot = (13, 15, 26, 6, 17, 29, 16, 24)
    ks = (np.uint32(k0), np.uint32(k1),
          np.uint32(k0) ^ np.uint32(k1) ^ np.uint32(0x1BD11BDA))
    x0 = (x0 + ks[0]).astype(np.uint32)
    x1 = (x1 + ks[1]).astype(np.uint32)
    for i in range(5):
        for r in rot[:4] if i % 2 == 0 else rot[4:]:
            x0 = (x0 + x1).astype(np.uint32)
            x1 = ((x1 << np.uint32(r)) | (x1 >> np.uint32(32 - r))).astype(np.uint32)
            x1 = x1 ^ x0
        x0 = (x0 + ks[(i + 1) % 3]).astype(np.uint32)
        x1 = (x1 + ks[(i + 2) % 3] + np.uint32(i + 1)).astype(np.uint32)
    return x0, x1


def _bernoulli_mask(seed, p, n):
    k0 = np.uint32(np.uint64(seed) >> np.uint64(32))
    k1 = np.uint32(np.uint64(seed) & np.uint64(0xFFFFFFFF))
    idx = np.arange(n, dtype=np.uint64)
    c1 = (idx >> np.uint64(32)).astype(np.uint32)
    c2 = (idx & np.uint64(0xFFFFFFFF)).astype(np.uint32)
    b1, b2 = _threefry2x32(k0, k1, c1, c2)
    bits = b1 ^ b2
    floats = ((bits >> np.uint32(9)) | np.uint32(0x3F800000)).view(np.float32)
    u = np.maximum(np.float32(0), floats - np.float32(1))
    return u < np.float32(p)


def _pack(idx):
    per = -(-len(idx) // _NW)
    per = -(-per // _C) * _C
    npad = per * _NW - len(idx)
    reps = -(-npad // len(idx))
    pad = np.tile(idx, reps)[:npad]
    return np.concatenate([idx, pad]).reshape(_NW, per // _C, _C)


def _mask_plan(n):
    if n not in _plan_cache:
        mask = _bernoulli_mask(_SEED, _Q, n)
        keep = np.nonzero(~mask)[0].astype(np.int32)
        zero = np.nonzero(mask)[0].astype(np.int32)
        _plan_cache[n] = (_pack(keep), _pack(zero))
    return _plan_cache[n]


def _sc_body(kc, zc, x_hbm, kidx_hbm, zidx_hbm, zeros_hbm, out_hbm,
             kidx_v, zidx_v, rowbuf, zbuf, gsem, ssem, zsem):
    nb = rowbuf.shape[0]
    w = lax.axis_index("s") * _NC + lax.axis_index("c")
    pk = pltpu.async_copy(kidx_hbm.at[w], kidx_v, gsem)
    pz = pltpu.async_copy(zidx_hbm.at[w], zidx_v, ssem)
    pb = pltpu.async_copy(zeros_hbm.at[w], zbuf, zsem)

    def gather(j):
        return pltpu.async_copy(x_hbm.at[kidx_v.at[j]], rowbuf.at[j % nb], gsem)

    def scatter(j):
        return pltpu.async_copy(rowbuf.at[j % nb], out_hbm.at[kidx_v.at[j]], ssem)

    def zero(j):
        return pltpu.async_copy(zbuf, out_hbm.at[zidx_v.at[j]], zsem)

    pk.wait()
    gd = {j: gather(j) for j in range(min(nb, kc))}
    pz.wait()
    pb.wait()
    zq = [zero(0), zero(1)]
    znext = 2
    sd = {}
    for j in range(kc):
        gd[j].wait()
        sd[j] = scatter(j)
        if j >= 1 and j + nb - 1 < kc:
            sd[j - 1].wait()
            gd[j + nb - 1] = gather(j + nb - 1)
        take = -(-(zc - znext) // max(1, kc - 1 - j))
        for _ in range(take):
            if znext < zc:
                zq.append(zero(znext))
                znext += 1
    while znext < zc:
        zq.append(zero(znext))
        znext += 1
    for j in sorted(sd):
        if not (1 <= j + 1 and j + nb < kc):
            sd[j].wait()
    for d in zq:
        d.wait()


def kernel(x):
    n, d = x.shape
    kidx, zidx = _mask_plan(n)
    kc, zc = kidx.shape[1], zidx.shape[1]
    zeros = jnp.zeros((_NW, _C, d), x.dtype)
    mesh = plsc.VectorSubcoreMesh(
        core_axis_name="c", subcore_axis_name="s",
        num_cores=_NC, num_subcores=_NS)
    body = functools.partial(_sc_body, kc, zc)
    return pl.kernel(
        body,
        out_type=jax.ShapeDtypeStruct((n, d), x.dtype),
        mesh=mesh,
        scratch_types=[
            pltpu.VMEM((kc, _C), jnp.int32),
            pltpu.VMEM((zc, _C), jnp.int32),
            pltpu.VMEM((6, _C, d), x.dtype),
            pltpu.VMEM((_C, d), x.dtype),
            pltpu.SemaphoreType.DMA,
            pltpu.SemaphoreType.DMA,
            pltpu.SemaphoreType.DMA,
        ],
    )(x, jnp.asarray(kidx), jnp.asarray(zidx), zeros)


# speedup vs baseline: 1.1335x; 1.0271x over previous
import functools
import numpy as np
import jax
import jax.numpy as jnp
from jax import lax
from jax.experimental import pallas as pl
from jax.experimental.pallas import tpu as pltpu
from jax.experimental.pallas import tpu_sc as plsc

_Q = 0.7
_SEED = 42
_C = 128
_NC = 2
_NS = 16
_NW = _NC * _NS

_plan_cache = {}


def _threefry2x32(k0, k1, x0, x1):
    rot = (13, 15, 26, 6, 17, 29, 16, 24)
    ks = (np.uint32(k0), np.uint32(k1),
          np.uint32(k0) ^ np.uint32(k1) ^ np.uint32(0x1BD11BDA))
    x0 = (x0 + ks[0]).astype(np.uint32)
    x1 = (x1 + ks[1]).astype(np.uint32)
    for i in range(5):
        for r in rot[:4] if i % 2 == 0 else rot[4:]:
            x0 = (x0 + x1).astype(np.uint32)
            x1 = ((x1 << np.uint32(r)) | (x1 >> np.uint32(32 - r))).astype(np.uint32)
            x1 = x1 ^ x0
        x0 = (x0 + ks[(i + 1) % 3]).astype(np.uint32)
        x1 = (x1 + ks[(i + 2) % 3] + np.uint32(i + 1)).astype(np.uint32)
    return x0, x1


def _bernoulli_mask(seed, p, n):
    k0 = np.uint32(np.uint64(seed) >> np.uint64(32))
    k1 = np.uint32(np.uint64(seed) & np.uint64(0xFFFFFFFF))
    idx = np.arange(n, dtype=np.uint64)
    c1 = (idx >> np.uint64(32)).astype(np.uint32)
    c2 = (idx & np.uint64(0xFFFFFFFF)).astype(np.uint32)
    b1, b2 = _threefry2x32(k0, k1, c1, c2)
    bits = b1 ^ b2
    floats = ((bits >> np.uint32(9)) | np.uint32(0x3F800000)).view(np.float32)
    u = np.maximum(np.float32(0), floats - np.float32(1))
    return u < np.float32(p)


def _pack(idx):
    q = -(-len(idx) // _NW)
    per = -(-q // _C) * _C
    npad = per * _NW - len(idx)
    reps = -(-npad // len(idx))
    pad = np.tile(idx, reps)[:npad]
    whole = np.concatenate([idx[:_NW * q], np.tile(idx, reps)[:_NW * q - len(idx)]])
    arr = np.full((_NW, per), 0, np.int32)
    arr[:, :q] = whole.reshape(_NW, q)
    if per > q:
        arr[:, q:] = arr[:, :per - q]
    return arr.reshape(_NW, per // _C, _C), q


def _mask_plan(n):
    if n not in _plan_cache:
        mask = _bernoulli_mask(_SEED, _Q, n)
        keep = np.nonzero(~mask)[0].astype(np.int32)
        zero = np.nonzero(mask)[0].astype(np.int32)
        _plan_cache[n] = (*_pack(keep), *_pack(zero))
    return _plan_cache[n]


def _sc_body(kc, krem, zc, zrem, x_hbm, kidx_hbm, zidx_hbm, zeros_hbm,
             out_hbm, kidx_v, zidx_v, rowbuf, zbuf, gsem, ssem, zsem):
    nb = rowbuf.shape[0]
    w = lax.axis_index("s") * _NC + lax.axis_index("c")
    pk = pltpu.async_copy(kidx_hbm.at[w], kidx_v, gsem)
    pz = pltpu.async_copy(zidx_hbm.at[w], zidx_v, ssem)
    pb = pltpu.async_copy(zeros_hbm.at[w], zbuf, zsem)

    def gather(j):
        return pltpu.async_copy(x_hbm.at[kidx_v.at[j]], rowbuf.at[j % nb], gsem)

    def scatter(j):
        return pltpu.async_copy(rowbuf.at[j % nb], out_hbm.at[kidx_v.at[j]], ssem)

    def zero(j):
        return pltpu.async_copy(zbuf, out_hbm.at[zidx_v.at[j]], zsem)

    pk.wait()
    gd = {j: gather(j) for j in range(min(nb, kc))}
    pz.wait()
    pb.wait()
    zq = [zero(0), zero(1)]
    znext = 2
    sd = {}
    for j in range(kc):
        gd[j].wait()
        sd[j] = scatter(j)
        if j >= 1 and j + nb - 1 < kc:
            sd[j - 1].wait()
            gd[j + nb - 1] = gather(j + nb - 1)
        take = -(-(zc - znext) // max(1, kc - 1 - j))
        for _ in range(take):
            if znext < zc:
                zq.append(zero(znext))
                znext += 1
    while znext < zc:
        zq.append(zero(znext))
        znext += 1
    if zrem:
        zq.append(pltpu.async_copy(
            zbuf.at[pl.ds(0, zrem)],
            out_hbm.at[zidx_v.at[zc, pl.ds(0, zrem)]], zsem))
    if krem:
        if kc >= nb:
            sd[kc - nb].wait()
            del sd[kc - nb]
        pltpu.async_copy(
            x_hbm.at[kidx_v.at[kc, pl.ds(0, krem)]],
            rowbuf.at[kc % nb, pl.ds(0, krem)], gsem).wait()
        sd[kc] = pltpu.async_copy(
            rowbuf.at[kc % nb, pl.ds(0, krem)],
            out_hbm.at[kidx_v.at[kc, pl.ds(0, krem)]], ssem)
    for j in sorted(sd):
        if not (1 <= j + 1 and j + nb < kc):
            sd[j].wait()
    for d in zq:
        d.wait()


def kernel(x):
    n, d = x.shape
    kidx, kq, zidx, zq = _mask_plan(n)
    kc, krem = kq // _C, kq % _C
    zc, zrem = zq // _C, zq % _C
    zeros = jnp.zeros((_NW, _C, d), x.dtype)
    mesh = plsc.VectorSubcoreMesh(
        core_axis_name="c", subcore_axis_name="s",
        num_cores=_NC, num_subcores=_NS)
    body = functools.partial(_sc_body, kc, krem, zc, zrem)
    return pl.kernel(
        body,
        out_type=jax.ShapeDtypeStruct((n, d), x.dtype),
        mesh=mesh,
        scratch_types=[
            pltpu.VMEM((kidx.shape[1], _C), jnp.int32),
            pltpu.VMEM((zidx.shape[1], _C), jnp.int32),
            pltpu.VMEM((6, _C, d), x.dtype),
            pltpu.VMEM((_C, d), x.dtype),
            pltpu.SemaphoreType.DMA,
            pltpu.SemaphoreType.DMA,
            pltpu.SemaphoreType.DMA,
        ],
    )(x, jnp.asarray(kidx), jnp.asarray(zidx), zeros)


# speedup vs baseline: 1.1656x; 1.0283x over previous
import functools
import numpy as np
import jax
import jax.numpy as jnp
from jax import lax
from jax.experimental import pallas as pl
from jax.experimental.pallas import tpu as pltpu
from jax.experimental.pallas import tpu_sc as plsc

_Q = 0.7
_SEED = 42
_C = 128
_NC = 2
_NS = 16
_NW = _NC * _NS

_plan_cache = {}


def _threefry2x32(k0, k1, x0, x1):
    rot = (13, 15, 26, 6, 17, 29, 16, 24)
    ks = (np.uint32(k0), np.uint32(k1),
          np.uint32(k0) ^ np.uint32(k1) ^ np.uint32(0x1BD11BDA))
    x0 = (x0 + ks[0]).astype(np.uint32)
    x1 = (x1 + ks[1]).astype(np.uint32)
    for i in range(5):
        for r in rot[:4] if i % 2 == 0 else rot[4:]:
            x0 = (x0 + x1).astype(np.uint32)
            x1 = ((x1 << np.uint32(r)) | (x1 >> np.uint32(32 - r))).astype(np.uint32)
            x1 = x1 ^ x0
        x0 = (x0 + ks[(i + 1) % 3]).astype(np.uint32)
        x1 = (x1 + ks[(i + 2) % 3] + np.uint32(i + 1)).astype(np.uint32)
    return x0, x1


def _bernoulli_mask(seed, p, n):
    k0 = np.uint32(np.uint64(seed) >> np.uint64(32))
    k1 = np.uint32(np.uint64(seed) & np.uint64(0xFFFFFFFF))
    idx = np.arange(n, dtype=np.uint64)
    c1 = (idx >> np.uint64(32)).astype(np.uint32)
    c2 = (idx & np.uint64(0xFFFFFFFF)).astype(np.uint32)
    b1, b2 = _threefry2x32(k0, k1, c1, c2)
    bits = b1 ^ b2
    floats = ((bits >> np.uint32(9)) | np.uint32(0x3F800000)).view(np.float32)
    u = np.maximum(np.float32(0), floats - np.float32(1))
    return u < np.float32(p)


def _pack(idx):
    q = -(-len(idx) // _NW)
    per = -(-q // _C) * _C
    npad = per * _NW - len(idx)
    reps = -(-npad // len(idx))
    pad = np.tile(idx, reps)[:npad]
    whole = np.concatenate([idx[:_NW * q], np.tile(idx, reps)[:_NW * q - len(idx)]])
    arr = np.full((_NW, per), 0, np.int32)
    arr[:, :q] = whole.reshape(_NW, q)
    if per > q:
        arr[:, q:] = arr[:, :per - q]
    return arr.reshape(_NW, per // _C, _C), q


def _mask_plan(n):
    if n not in _plan_cache:
        mask = _bernoulli_mask(_SEED, _Q, n)
        keep = np.nonzero(~mask)[0].astype(np.int32)
        zero = np.nonzero(mask)[0].astype(np.int32)
        _plan_cache[n] = (*_pack(keep), *_pack(zero))
    return _plan_cache[n]


def _sc_body(kc, krem, zc, zrem, x_hbm, idx_hbm, zeros_hbm,
             out_hbm, idx_v, rowbuf, zbuf, gsem, ssem, zsem):
    nb = rowbuf.shape[0]
    kci = kc + (1 if krem else 0)
    w = lax.axis_index("s") * _NC + lax.axis_index("c")
    pk = pltpu.async_copy(idx_hbm.at[w], idx_v, gsem)
    pb = pltpu.async_copy(zeros_hbm.at[w], zbuf, zsem)

    def gather(j):
        return pltpu.async_copy(x_hbm.at[idx_v.at[j]], rowbuf.at[j % nb], gsem)

    def scatter(j):
        return pltpu.async_copy(rowbuf.at[j % nb], out_hbm.at[idx_v.at[j]], ssem)

    def zero(j):
        return pltpu.async_copy(zbuf, out_hbm.at[idx_v.at[kci + j]], zsem)

    pk.wait()
    gd = {j: gather(j) for j in range(min(nb, kc))}
    pb.wait()
    zq = [zero(0), zero(1)]
    znext = 2
    sd = {}
    for j in range(kc):
        gd[j].wait()
        sd[j] = scatter(j)
        if j >= 1 and j + nb - 1 < kc:
            sd[j - 1].wait()
            gd[j + nb - 1] = gather(j + nb - 1)
        take = -(-(zc - znext) // max(1, kc - 1 - j))
        for _ in range(take):
            if znext < zc:
                zq.append(zero(znext))
                znext += 1
    while znext < zc:
        zq.append(zero(znext))
        znext += 1
    if zrem:
        zq.append(pltpu.async_copy(
            zbuf.at[pl.ds(0, zrem)],
            out_hbm.at[idx_v.at[kci + zc, pl.ds(0, zrem)]], zsem))
    if krem:
        if kc >= nb:
            sd[kc - nb].wait()
            del sd[kc - nb]
        pltpu.async_copy(
            x_hbm.at[idx_v.at[kc, pl.ds(0, krem)]],
            rowbuf.at[kc % nb, pl.ds(0, krem)], gsem).wait()
        sd[kc] = pltpu.async_copy(
            rowbuf.at[kc % nb, pl.ds(0, krem)],
            out_hbm.at[idx_v.at[kc, pl.ds(0, krem)]], ssem)
    for j in sorted(sd):
        if not (1 <= j + 1 and j + nb < kc):
            sd[j].wait()
    for d in zq:
        d.wait()


def kernel(x):
    n, d = x.shape
    kidx, kq, zidx, zq = _mask_plan(n)
    kc, krem = kq // _C, kq % _C
    zc, zrem = zq // _C, zq % _C
    idx = np.concatenate([kidx, zidx], axis=1)
    zeros = np.zeros((_NW, _C, d), np.float32)
    mesh = plsc.VectorSubcoreMesh(
        core_axis_name="c", subcore_axis_name="s",
        num_cores=_NC, num_subcores=_NS)
    body = functools.partial(_sc_body, kc, krem, zc, zrem)
    return pl.kernel(
        body,
        out_type=jax.ShapeDtypeStruct((n, d), x.dtype),
        mesh=mesh,
        scratch_types=[
            pltpu.VMEM((idx.shape[1], _C), jnp.int32),
            pltpu.VMEM((6, _C, d), x.dtype),
            pltpu.VMEM((_C, d), x.dtype),
            pltpu.SemaphoreType.DMA,
            pltpu.SemaphoreType.DMA,
            pltpu.SemaphoreType.DMA,
        ],
    )(x, jnp.asarray(idx), jnp.asarray(zeros))


# speedup vs baseline: 1.2009x; 1.0303x over previous
import functools
import numpy as np
import jax
import jax.numpy as jnp
from jax import lax
from jax.experimental import pallas as pl
from jax.experimental.pallas import tpu as pltpu
from jax.experimental.pallas import tpu_sc as plsc

_Q = 0.7
_SEED = 42
_C = 128
_NC = 2
_NS = 16
_NW = _NC * _NS

_plan_cache = {}


def _threefry2x32(k0, k1, x0, x1):
    rot = (13, 15, 26, 6, 17, 29, 16, 24)
    ks = (np.uint32(k0), np.uint32(k1),
          np.uint32(k0) ^ np.uint32(k1) ^ np.uint32(0x1BD11BDA))
    x0 = (x0 + ks[0]).astype(np.uint32)
    x1 = (x1 + ks[1]).astype(np.uint32)
    for i in range(5):
        for r in rot[:4] if i % 2 == 0 else rot[4:]:
            x0 = (x0 + x1).astype(np.uint32)
            x1 = ((x1 << np.uint32(r)) | (x1 >> np.uint32(32 - r))).astype(np.uint32)
            x1 = x1 ^ x0
        x0 = (x0 + ks[(i + 1) % 3]).astype(np.uint32)
        x1 = (x1 + ks[(i + 2) % 3] + np.uint32(i + 1)).astype(np.uint32)
    return x0, x1


def _bernoulli_mask(seed, p, n):
    k0 = np.uint32(np.uint64(seed) >> np.uint64(32))
    k1 = np.uint32(np.uint64(seed) & np.uint64(0xFFFFFFFF))
    idx = np.arange(n, dtype=np.uint64)
    c1 = (idx >> np.uint64(32)).astype(np.uint32)
    c2 = (idx & np.uint64(0xFFFFFFFF)).astype(np.uint32)
    b1, b2 = _threefry2x32(k0, k1, c1, c2)
    bits = b1 ^ b2
    floats = ((bits >> np.uint32(9)) | np.uint32(0x3F800000)).view(np.float32)
    u = np.maximum(np.float32(0), floats - np.float32(1))
    return u < np.float32(p)


def _pack(idx):
    q = -(-len(idx) // _NW)
    per = -(-q // _C) * _C
    npad = per * _NW - len(idx)
    reps = -(-npad // len(idx))
    pad = np.tile(idx, reps)[:npad]
    whole = np.concatenate([idx[:_NW * q], np.tile(idx, reps)[:_NW * q - len(idx)]])
    arr = np.full((_NW, per), 0, np.int32)
    arr[:, :q] = whole.reshape(_NW, q)
    if per > q:
        arr[:, q:] = arr[:, :per - q]
    return arr.reshape(_NW, per // _C, _C), q


def _mask_plan(n):
    if n not in _plan_cache:
        mask = _bernoulli_mask(_SEED, _Q, n)
        keep = np.nonzero(~mask)[0].astype(np.int32)
        zero = np.nonzero(mask)[0].astype(np.int32)
        _plan_cache[n] = (*_pack(keep), *_pack(zero))
    return _plan_cache[n]


def _sc_body(kc, krem, zc, zrem, x_hbm, idx_hbm,
             out_hbm, idx_v, rowbuf, zbuf, gsem, ssem, zsem):
    nb = rowbuf.shape[0]
    kci = kc + (1 if krem else 0)
    w = lax.axis_index("s") * _NC + lax.axis_index("c")
    pk = pltpu.async_copy(idx_hbm.at[w], idx_v, gsem)

    zv = jnp.zeros((16,), zbuf.dtype)

    def _zrow(i, _):
        for k in range(zbuf.shape[1] // 16):
            zbuf[i, pl.ds(k * 16, 16)] = zv
        return _

    lax.fori_loop(0, zbuf.shape[0], _zrow, 0)

    def gather(j):
        return pltpu.async_copy(x_hbm.at[idx_v.at[j]], rowbuf.at[j % nb], gsem)

    def scatter(j):
        return pltpu.async_copy(rowbuf.at[j % nb], out_hbm.at[idx_v.at[j]], ssem)

    def zero(j):
        return pltpu.async_copy(zbuf, out_hbm.at[idx_v.at[kci + j]], zsem)

    pk.wait()
    gd = {j: gather(j) for j in range(min(nb, kc))}
    zq = [zero(0), zero(1)]
    znext = 2
    sd = {}
    for j in range(kc):
        gd[j].wait()
        sd[j] = scatter(j)
        if j >= 1 and j + nb - 1 < kc:
            sd[j - 1].wait()
            gd[j + nb - 1] = gather(j + nb - 1)
        take = -(-(zc - znext) // max(1, kc - 1 - j))
        for _ in range(take):
            if znext < zc:
                zq.append(zero(znext))
                znext += 1
    while znext < zc:
        zq.append(zero(znext))
        znext += 1
    if zrem:
        zq.append(pltpu.async_copy(
            zbuf.at[pl.ds(0, zrem)],
            out_hbm.at[idx_v.at[kci + zc, pl.ds(0, zrem)]], zsem))
    if krem:
        if kc >= nb:
            sd[kc - nb].wait()
            del sd[kc - nb]
        pltpu.async_copy(
            x_hbm.at[idx_v.at[kc, pl.ds(0, krem)]],
            rowbuf.at[kc % nb, pl.ds(0, krem)], gsem).wait()
        sd[kc] = pltpu.async_copy(
            rowbuf.at[kc % nb, pl.ds(0, krem)],
            out_hbm.at[idx_v.at[kc, pl.ds(0, krem)]], ssem)
    for j in sorted(sd):
        if not (1 <= j + 1 and j + nb < kc):
            sd[j].wait()
    for d in zq:
        d.wait()


def kernel(x):
    n, d = x.shape
    kidx, kq, zidx, zq = _mask_plan(n)
    kc, krem = kq // _C, kq % _C
    zc, zrem = zq // _C, zq % _C
    idx = np.concatenate([kidx, zidx], axis=1)
    mesh = plsc.VectorSubcoreMesh(
        core_axis_name="c", subcore_axis_name="s",
        num_cores=_NC, num_subcores=_NS)
    body = functools.partial(_sc_body, kc, krem, zc, zrem)
    return pl.kernel(
        body,
        out_type=jax.ShapeDtypeStruct((n, d), x.dtype),
        mesh=mesh,
        scratch_types=[
            pltpu.VMEM((idx.shape[1], _C), jnp.int32),
            pltpu.VMEM((6, _C, d), x.dtype),
            pltpu.VMEM((_C, d), x.dtype),
            pltpu.SemaphoreType.DMA,
            pltpu.SemaphoreType.DMA,
            pltpu.SemaphoreType.DMA,
        ],
    )(x, jnp.asarray(idx))
